# baseline (speedup 1.0000x reference)
import jax
import jax.numpy as jnp
from jax.experimental import pallas as pl

POOL_SIZES = [25000, 6250, 1600, 400, 400]
VOX = 8
K = 5


def _fc_kernel(h_ref, w_ref, b_ref, o_ref):
    o_ref[...] = h_ref[...] @ w_ref[...] + b_ref[...]


def _fc(h, w, b):
    return pl.pallas_call(
        _fc_kernel,
        out_shape=jax.ShapeDtypeStruct((h.shape[0], w.shape[1]), jnp.float32),
    )(h, w, b[None, :])


def _conv(x, pos, ei, p):
    src, dst = ei[0], ei[1]
    pseudo = pos[dst] - pos[src]
    alpha = jax.nn.softmax(pseudo @ p["Bmap"], axis=-1)
    msg = jnp.einsum("ek,kio,ei->eo", alpha, p["Wk"], x[src])
    n = x.shape[0]
    agg = jax.ops.segment_sum(msg, dst, num_segments=n)
    cnt = jax.ops.segment_sum(jnp.ones((dst.shape[0],), x.dtype), dst, num_segments=n)
    agg = agg / jnp.maximum(cnt, 1.0)[:, None]
    out = x @ p["Wroot"] + agg + p["bias"]
    closs = -jnp.mean(jnp.sum(alpha * jnp.log(alpha + 1e-12), axis=-1))
    return jax.nn.elu(out), closs


def _pool(x, pos, cluster, s):
    xp = jax.ops.segment_max(x, cluster, num_segments=s)
    xp = jnp.where(jnp.isfinite(xp), xp, 0.0)
    cnt = jax.ops.segment_sum(jnp.ones((cluster.shape[0],), x.dtype), cluster, num_segments=s)
    posp = jax.ops.segment_sum(pos, cluster, num_segments=s) / jnp.maximum(cnt, 1.0)[:, None]
    return jnp.concatenate([xp, posp], axis=1), posp


def kernel(x, pos, edge_index0, edge_index1, edge_index2, edge_index3, edge_index4,
           cluster1, cluster2, cluster3, cluster4, cluster5, params):
    eis = [edge_index0, edge_index1, edge_index2, edge_index3, edge_index4]
    clusters = [cluster1, cluster2, cluster3, cluster4, cluster5]
    closs = jnp.asarray(0.0, dtype=jnp.float32)
    for l in range(5):
        x, c = _conv(x, pos, eis[l], params[f"conv{l + 1}"])
        closs = closs + c
        x, pos = _pool(x, pos, clusters[l], POOL_SIZES[l])
    h = x.reshape(-1, VOX * 47)
    logits = _fc(h, params["fcW"], params["fcb"])
    return jax.nn.log_softmax(logits, axis=1), closs


# speedup vs baseline: 6.7883x; 6.7883x over previous
import functools
import math

import jax
import jax.numpy as jnp
from jax import lax
from jax.experimental import pallas as pl
from jax.experimental.pallas import tpu as pltpu
from jax.experimental.pallas import tpu_sc as plsc

N0, N1, N2, N3, N4 = 100000, 25000, 6250, 1600, 400
B, VOX = 50, 8
K = 5
DIMS = [(1, 12), (15, 20), (23, 28), (31, 36), (39, 44)]
SIZES = [N0, N1, N2, N3, N4]
POOL_SIZES = [N1, N2, N3, N4, B * VOX]
DEG = 16

NC, NS, LANES = 2, 16, 16
NW = NC * NS
_INTERPRET = False

CONV_BN = [2000, 1000, 3128, 1600, 400]

CONV_CFG = [
    dict(J=8, mode="outer"),
    dict(J=4, mode="ymsg"),
    dict(J=4, mode="ymsg"),
    dict(J=2, mode="ymsg"),
    dict(J=1, mode="ymsg"),
]

LN2 = 0.6931471805599453
SQRT2 = 1.4142135623730951


def _vlog(s):
    bits = plsc.bitcast(s, jnp.int32)
    e = (lax.shift_right_logical(bits, 23) & 0xFF) - 127
    m_bits = (bits & 0x7FFFFF) | 0x3F800000
    m = plsc.bitcast(m_bits, jnp.float32)
    big = m > SQRT2
    m = jnp.where(big, m * 0.5, m)
    ef = e.astype(jnp.float32) + jnp.where(big, 1.0, 0.0)
    t = (m - 1.0) / (m + 1.0)
    t2 = t * t
    p = 1.0 + t2 * (1.0 / 3.0 + t2 * (1.0 / 5.0 + t2 * (1.0 / 7.0)))
    return ef * LN2 + 2.0 * t * p


def _conv_sc_level(level, n, ci, co, e_real):
    cfg = CONV_CFG[level]
    J = cfg["J"]
    mode = cfg["mode"]
    C = J * 128
    TW = ci if mode == "outer" else K * co
    W = (K * ci + 1) if mode == "outer" else (co + 1)
    mw = math.ceil(e_real / (NW * C))
    e_pad = NW * C * mw
    n_pad = math.ceil(n / 8) * 8
    kc = math.ceil(n_pad / (NS * 128))

    mesh = plsc.VectorSubcoreMesh(
        core_axis_name="c", subcore_axis_name="s", num_cores=NC, num_subcores=NS)

    scratch = []
    scratch.append(pltpu.VMEM_SHARED((n_pad, W), jnp.float32))
    for _ in range(J):
        scratch.append(pltpu.VMEM((128,), jnp.int32))
    for _ in range(J):
        scratch.append(pltpu.VMEM((128,), jnp.int32))
    scratch.append(pltpu.VMEM((C, 3), jnp.float32))
    scratch.append(pltpu.VMEM((C, 3), jnp.float32))
    scratch.append(pltpu.VMEM((C, TW), jnp.float32))
    scratch.append(pltpu.VMEM((C, W), jnp.float32))
    scratch.append(pltpu.VMEM((64, W), jnp.float32))
    scratch.append(pltpu.VMEM((16,), jnp.float32))
    scratch.append(pltpu.VMEM((16,), jnp.float32))
    scratch.append(pltpu.SemaphoreType.DMA)

    def body(src_hbm, dst_hbm, tab_hbm, pos_hbm, bmap_hbm, z_hbm, s2_out, h_out,
             *scr):
        s_sh = scr[0]
        sidx = list(scr[1:1 + J])
        didx = list(scr[1 + J:1 + 2 * J])
        ps2, pd2, tg2, rows2, zbuf, hbuf, bmap_v = scr[1 + 2 * J:1 + 2 * J + 7]
        sem = scr[-1]

        cid = lax.axis_index("c")
        sid = lax.axis_index("s")
        wid = sid * NC + cid

        pltpu.sync_copy(bmap_hbm, bmap_v)

        pltpu.sync_copy(z_hbm, zbuf)
        n_zc = math.ceil(n_pad / (NS * 64))
        zrows = n_zc * 64

        def _zs(i, _):
            r0 = jnp.minimum(sid * zrows + i * 64, n_pad - 64)
            r0 = jnp.maximum(r0, 0)
            pltpu.sync_copy(zbuf, s_sh.at[pl.ds(r0, 64)])
            return 0
        lax.fori_loop(0, n_zc, _zs, 0)
        plsc.subcore_barrier()

        iota = lax.iota(jnp.int32, 16)
        bvec = bmap_v[...]
        bm = [[bvec[i * K + k] for k in range(K)] for i in range(3)]

        def chunk_body(m, hacc):
            chunk_off = (wid * mw + m) * C
            for j in range(J):
                pltpu.sync_copy(src_hbm.at[pl.ds(chunk_off + j * 128, 128)], sidx[j])
                pltpu.sync_copy(dst_hbm.at[pl.ds(chunk_off + j * 128, 128)], didx[j])
            copies = []
            for j in range(J):
                copies.append(pltpu.async_copy(
                    pos_hbm.at[sidx[j]], ps2.at[pl.ds(j * 128, 128)], sem))
                copies.append(pltpu.async_copy(
                    pos_hbm.at[didx[j]], pd2.at[pl.ds(j * 128, 128)], sem))
                copies.append(pltpu.async_copy(
                    tab_hbm.at[sidx[j]], tg2.at[pl.ds(j * 128, 128)], sem))
            for cp in copies:
                cp.wait()

            for j in range(J):
                def v_body(v8, h_in, j=j):
                    rl = v8 * 16 + iota
                    fr = j * 128 + rl
                    eg = chunk_off + fr
                    vf = jnp.where(eg < e_real, 1.0, 0.0)
                    p0 = plsc.load_gather(ps2, [fr, jnp.full((16,), 0, jnp.int32)])
                    p1 = plsc.load_gather(ps2, [fr, jnp.full((16,), 1, jnp.int32)])
                    p2 = plsc.load_gather(ps2, [fr, jnp.full((16,), 2, jnp.int32)])
                    q0 = plsc.load_gather(pd2, [fr, jnp.full((16,), 0, jnp.int32)])
                    q1 = plsc.load_gather(pd2, [fr, jnp.full((16,), 1, jnp.int32)])
                    q2 = plsc.load_gather(pd2, [fr, jnp.full((16,), 2, jnp.int32)])
                    u0, u1, u2 = q0 - p0, q1 - p1, q2 - p2
                    z = [u0 * bm[0][k] + u1 * bm[1][k] + u2 * bm[2][k]
                         for k in range(K)]
                    zm = z[0]
                    for k in range(1, K):
                        zm = jnp.maximum(zm, z[k])
                    ez = [jnp.exp(zk - zm) for zk in z]
                    ssum = ez[0]
                    for k in range(1, K):
                        ssum = ssum + ez[k]
                    inv = 1.0 / ssum
                    alpha = [ek * inv for ek in ez]
                    dot = alpha[0] * (z[0] - zm)
                    for k in range(1, K):
                        dot = dot + alpha[k] * (z[k] - zm)
                    h_new = h_in + vf * (_vlog(ssum) - dot)

                    def col(c):
                        return jnp.full((16,), c, jnp.int32)

                    if mode == "outer":
                        xs = [plsc.load_gather(tg2, [fr, col(i)])
                              for i in range(ci)]
                        for k in range(K):
                            avk = alpha[k] * vf
                            for i in range(ci):
                                plsc.store_scatter(
                                    rows2, [fr, col(k * ci + i)], avk * xs[i])
                    else:
                        for o in range(co):
                            acc = alpha[0] * plsc.load_gather(tg2, [fr, col(o)])
                            for k in range(1, K):
                                acc = acc + alpha[k] * plsc.load_gather(
                                    tg2, [fr, col(k * co + o)])
                            plsc.store_scatter(rows2, [fr, col(o)], acc * vf)
                    plsc.store_scatter(rows2, [fr, col(W - 1)], vf)
                    return h_new
                hacc = lax.fori_loop(0, 8, v_body, hacc)
            for j in range(J):
                pltpu.sync_copy(rows2.at[pl.ds(j * 128, 128)],
                                s_sh.at[didx[j]], add=True)
            return hacc

        hacc = lax.fori_loop(0, mw, chunk_body, jnp.zeros((16,), jnp.float32))
        hbuf[...] = hacc
        pltpu.sync_copy(hbuf, h_out.at[wid])
        plsc.subcore_barrier()

        def _out(i, _):
            r0 = jnp.minimum(sid * (kc * 128) + i * 128, n_pad - 128)
            pltpu.sync_copy(s_sh.at[pl.ds(r0, 128)], s2_out.at[cid, pl.ds(r0, 128)])
            return 0
        lax.fori_loop(0, kc, _out, 0)

    kern = pl.kernel(
        body,
        out_type=[
            jax.ShapeDtypeStruct((2, n_pad, W), jnp.float32),
            jax.ShapeDtypeStruct((NW, 16), jnp.float32),
        ],
        mesh=mesh,
        scratch_types=scratch,
        compiler_params=pltpu.CompilerParams(
            needs_layout_passes=False, use_tc_tiling_on_sc=False),
        interpret=_INTERPRET,
    )

    def run(ei, table, pos, bmap):
        srcF = jnp.pad(ei[0], (0, e_pad - e_real))
        dstF = jnp.pad(ei[1], (0, e_pad - e_real))
        bmap16 = jnp.pad(bmap.reshape(15), (0, 1))
        zeros = jnp.zeros((64, W), jnp.float32)
        return kern(srcF, dstF, table, pos, bmap16, zeros)

    return run


_CONV_RUNNERS = {}


def _get_conv_runner(level, n, ci, co, e_real):
    key = (level, n, ci, co, e_real)
    if key not in _CONV_RUNNERS:
        _CONV_RUNNERS[key] = _conv_sc_level(level, n, ci, co, e_real)
    return _CONV_RUNNERS[key]


def _conv_tc_post(mode, s2, x, wk2, wroot, bias, bn):
    n, ci = x.shape
    co = wroot.shape[1]
    n_pad = s2.shape[1]
    x = jnp.pad(x, ((0, n_pad - n), (0, 0)))
    w = s2.shape[2]
    grid = n_pad // bn

    def body(s2_ref, x_ref, wk2_ref, wroot_ref, bias_ref, o_ref):
        s = s2_ref[0] + s2_ref[1]
        if mode == "outer":
            agg = jnp.dot(s[:, :K * ci], wk2_ref[...],
                          preferred_element_type=jnp.float32)
            cnt = s[:, K * ci]
        else:
            agg = s[:, :co]
            cnt = s[:, co]
        agg = agg / jnp.maximum(cnt, 1.0)[:, None]
        out = jnp.dot(x_ref[...], wroot_ref[...],
                      preferred_element_type=jnp.float32)
        out = out + agg + bias_ref[0:1]
        o_ref[...] = jnp.where(out > 0, out, jnp.exp(jnp.minimum(out, 0.0)) - 1.0)

    res = pl.pallas_call(
        body,
        grid=(grid,),
        in_specs=[
            pl.BlockSpec((2, bn, w), lambda i: (0, i, 0)),
            pl.BlockSpec((bn, ci), lambda i: (i, 0)),
            pl.BlockSpec(wk2.shape, lambda i: (0, 0)),
            pl.BlockSpec(wroot.shape, lambda i: (0, 0)),
            pl.BlockSpec((8, co), lambda i: (0, 0)),
        ],
        out_specs=pl.BlockSpec((bn, co), lambda i: (i, 0)),
        out_shape=jax.ShapeDtypeStruct((n_pad, co), jnp.float32),
    )(s2, x, wk2, wroot, jnp.broadcast_to(bias[None, :], (8, co)))
    return res[:n]


def _conv(level, x, pos, ei, p, table):
    n, ci = x.shape
    co = p["Wroot"].shape[1]
    e_real = ei.shape[1]
    mode = CONV_CFG[level]["mode"]
    run = _get_conv_runner(level, n, ci, co, e_real)
    s2, hout = run(ei, table, pos, p["Bmap"])
    wk2 = p["Wk"].reshape(K * ci, co)
    out = _conv_tc_post(mode, s2, x, wk2, p["Wroot"], p["bias"], CONV_BN[level])
    return out, hout


POOL_CFG = [
    dict(G=8, R=4, Jp=4),
    dict(G=16, R=2, Jp=4),
    dict(G=16, R=2, Jp=2),
    dict(G=32, R=1, Jp=1),
    dict(G=32, R=1, Jp=1),
]

NEG_BIG = -3.0e38


def _pool_sc_level(level, nn, s_out, f):
    cfg = POOL_CFG[level]
    G, R, Jp = cfg["G"], cfg["R"], cfg["Jp"]
    Cp = Jp * 128
    s_pad = math.ceil(s_out / 8) * 8
    rs = math.ceil(math.ceil(s_pad / R) / 8) * 8
    mp = math.ceil(nn / (G * Cp))
    kc4 = math.ceil(s_pad / (NS * 128))

    mesh = plsc.VectorSubcoreMesh(
        core_axis_name="c", subcore_axis_name="s", num_cores=NC, num_subcores=NS)

    scratch = [
        pltpu.VMEM_SHARED((s_pad, 4), jnp.float32),
        pltpu.VMEM((rs, f), jnp.float32),
        pltpu.VMEM((Cp,), jnp.int32),
        pltpu.VMEM((Cp, f), jnp.float32),
        pltpu.VMEM((Cp, 3), jnp.float32),
        pltpu.VMEM((Cp, 4), jnp.float32),
        pltpu.VMEM((64, 4), jnp.float32),
    ]
    for _ in range(Jp):
        scratch.append(pltpu.VMEM((128,), jnp.int32))
    scratch.append(pltpu.SemaphoreType.DMA)

    def body(x_hbm, pos_hbm, cl_hbm, neg_hbm, z4_hbm, xp_out, p2_out, *scr):
        p_sh, table, clb, xb, posb, rows4, z4buf = scr[:7]
        clidx = list(scr[7:7 + Jp])
        sem = scr[-1]

        cid = lax.axis_index("c")
        sid = lax.axis_index("s")
        wid = sid * NC + cid
        g = wid // R
        r = wid - g * R
        rbase = r * rs

        n_tc = math.ceil(rs / 64)
        pltpu.sync_copy(z4_hbm, z4buf)

        def _ti(i, _):
            r0 = jnp.maximum(jnp.minimum(i * 64, rs - 64), 0)
            pltpu.sync_copy(neg_hbm, table.at[pl.ds(r0, 64)])
            return 0
        lax.fori_loop(0, n_tc, _ti, 0)

        n_zc = math.ceil(s_pad / (NS * 64))
        zrows = n_zc * 64

        def _zs(i, _):
            r0 = jnp.maximum(jnp.minimum(sid * zrows + i * 64, s_pad - 64), 0)
            pltpu.sync_copy(z4buf, p_sh.at[pl.ds(r0, 64)])
            return 0
        lax.fori_loop(0, n_zc, _zs, 0)
        plsc.subcore_barrier()

        iota = lax.iota(jnp.int32, 16)

        def col(c):
            return jnp.full((16,), c, jnp.int32)

        def chunk_body(m, _):
            off = (g * mp + m) * Cp
            pltpu.sync_copy(cl_hbm.at[pl.ds(off, Cp)], clb)
            pltpu.sync_copy(x_hbm.at[pl.ds(off, Cp)], xb)

            for j in range(Jp):
                def v_body(v8, _, j=j):
                    rl = j * 128 + v8 * 16 + iota
                    nidx = off + rl
                    valid = nidx < nn
                    cl = clb[pl.ds(j * 128 + v8 * 16, 16)]
                    loc = cl - rbase
                    msk = valid & (loc >= 0) & (loc < rs)
                    lcl = jnp.clip(loc, 0, rs - 1)
                    for ff in range(f):
                        xv = plsc.load_gather(xb, [rl, col(ff)])

                        def cond(carry):
                            return jnp.max(carry.astype(jnp.int32)) > 0

                        def it(carry):
                            cur = plsc.load_gather(table, [lcl, col(ff)])
                            new = jnp.maximum(cur, xv)
                            plsc.store_scatter(table, [lcl, col(ff)], new,
                                               mask=carry)
                            chk = plsc.load_gather(table, [lcl, col(ff)])
                            return carry & (chk < xv)

                        lax.while_loop(cond, it, msk)
                    return 0
                lax.fori_loop(0, 8, v_body, 0)

            @pl.when(r == 0)
            def _pos_part():
                pltpu.sync_copy(pos_hbm.at[pl.ds(off, Cp)], posb)
                for j in range(Jp):
                    def p_body(v8, _, j=j):
                        rl = j * 128 + v8 * 16 + iota
                        vf = jnp.where(off + rl < nn, 1.0, 0.0)
                        for cc in range(3):
                            pv = plsc.load_gather(posb, [rl, col(cc)])
                            plsc.store_scatter(rows4, [rl, col(cc)], pv * vf)
                        plsc.store_scatter(rows4, [rl, col(3)], vf)
                        return 0
                    lax.fori_loop(0, 8, p_body, 0)
                for j in range(Jp):
                    pltpu.sync_copy(cl_hbm.at[pl.ds(off + j * 128, 128)],
                                    clidx[j])
                    pltpu.sync_copy(rows4.at[pl.ds(j * 128, 128)],
                                    p_sh.at[clidx[j]], add=True)
            return 0

        lax.fori_loop(0, mp, chunk_body, 0)
        plsc.subcore_barrier()

        n_oc = math.ceil(rs / 128)

        def _ox(i, _):
            lo = jnp.minimum(i * 128, rs - 128)
            lo = jnp.minimum(lo, s_pad - 128 - rbase)
            lo = jnp.maximum(lo, 0)
            pltpu.sync_copy(table.at[pl.ds(lo, 128)],
                            xp_out.at[g, pl.ds(rbase + lo, 128)])
            return 0
        lax.fori_loop(0, n_oc, _ox, 0)

        def _op(i, _):
            r0 = jnp.maximum(jnp.minimum(sid * (kc4 * 128) + i * 128,
                                         s_pad - 128), 0)
            pltpu.sync_copy(p_sh.at[pl.ds(r0, 128)],
                            p2_out.at[cid, pl.ds(r0, 128)])
            return 0
        lax.fori_loop(0, kc4, _op, 0)

    kern = pl.kernel(
        body,
        out_type=[
            jax.ShapeDtypeStruct((G, s_pad, f), jnp.float32),
            jax.ShapeDtypeStruct((2, s_pad, 4), jnp.float32),
        ],
        mesh=mesh,
        scratch_types=scratch,
        compiler_params=pltpu.CompilerParams(
            needs_layout_passes=False, use_tc_tiling_on_sc=False),
        interpret=_INTERPRET,
    )

    def run(x, pos, cl):
        nn_pad = G * mp * Cp
        xp_ = jnp.pad(x, ((0, nn_pad - nn), (0, 0)))
        posp_ = jnp.pad(pos, ((0, nn_pad - nn), (0, 0)))
        clp_ = jnp.pad(cl, (0, nn_pad - nn))
        neg = jnp.full((64, f), NEG_BIG, jnp.float32)
        z4 = jnp.zeros((64, 4), jnp.float32)
        return kern(xp_, posp_, clp_, neg, z4)

    return run


_POOL_RUNNERS = {}


def _get_pool_runner(level, nn, s_out, f):
    key = (level, nn, s_out, f)
    if key not in _POOL_RUNNERS:
        _POOL_RUNNERS[key] = _pool_sc_level(level, nn, s_out, f)
    return _POOL_RUNNERS[key]


def _pool_tc_post(xp_g, p2, wkp, s, bn):
    g_, s_pad, f = xp_g.shape
    yw = 0 if wkp is None else wkp.shape[1]
    grid = s_pad // bn

    def body(*refs):
        if wkp is None:
            xp_ref, p2_ref, x_ref, pos_ref = refs
        else:
            xp_ref, p2_ref, wkp_ref, x_ref, pos_ref, y_ref = refs
        xp = xp_ref[0]
        for gg in range(1, g_):
            xp = jnp.maximum(xp, xp_ref[gg])
        xp = jnp.where(xp > NEG_BIG / 2, xp, 0.0)
        p = p2_ref[0] + p2_ref[1]
        cnt = p[:, 3]
        posp = p[:, :3] / jnp.maximum(cnt, 1.0)[:, None]
        xnew = jnp.concatenate([xp, posp], axis=1)
        x_ref[...] = xnew
        pos_ref[...] = posp
        if wkp is not None:
            y_ref[...] = jnp.dot(xnew, wkp_ref[...],
                                 preferred_element_type=jnp.float32)

    in_specs = [
        pl.BlockSpec((g_, bn, f), lambda i: (0, i, 0)),
        pl.BlockSpec((2, bn, 4), lambda i: (0, i, 0)),
    ]
    out_shape = [
        jax.ShapeDtypeStruct((s_pad, f + 3), jnp.float32),
        jax.ShapeDtypeStruct((s_pad, 3), jnp.float32),
    ]
    out_specs = [
        pl.BlockSpec((bn, f + 3), lambda i: (i, 0)),
        pl.BlockSpec((bn, 3), lambda i: (i, 0)),
    ]
    args = [xp_g, p2]
    if wkp is not None:
        in_specs.append(pl.BlockSpec(wkp.shape, lambda i: (0, 0)))
        out_shape.append(jax.ShapeDtypeStruct((s_pad, yw), jnp.float32))
        out_specs.append(pl.BlockSpec((bn, yw), lambda i: (i, 0)))
        args.append(wkp)
    res = pl.pallas_call(
        body, grid=(grid,), in_specs=in_specs, out_specs=out_specs,
        out_shape=out_shape,
    )(*args)
    xnew = res[0][:s]
    posnew = res[1][:s]
    ynext = res[2][:s] if wkp is not None else None
    return xnew, posnew, ynext


POOL_BN = [1000, 272, 400, 400, 400]


def _pool(level, x, pos, cluster, s, wkp):
    nn, f = x.shape
    run = _get_pool_runner(level, nn, s, f)
    xp_g, p2 = run(x, pos, cluster)
    return _pool_tc_post(xp_g, p2, wkp, s, POOL_BN[level])


def _final_tc(h, w, b, hstack):
    ecounts = [SIZES[l] * DEG for l in range(5)]

    def body(h_ref, w_ref, b_ref, hs_ref, ls_ref, closs_ref):
        logits = jnp.dot(h_ref[...], w_ref[...],
                         preferred_element_type=jnp.float32) + b_ref[...]
        m = jnp.max(logits, axis=1, keepdims=True)
        sh = logits - m
        lse = jnp.log(jnp.sum(jnp.exp(sh), axis=1, keepdims=True))
        ls_ref[...] = sh - lse
        c = jnp.float32(0.0)
        for l in range(5):
            c = c + jnp.sum(hs_ref[l]) / ecounts[l]
        closs_ref[...] = jnp.full((1, 1), 1.0, jnp.float32) * c

    return pl.pallas_call(
        body,
        out_shape=[
            jax.ShapeDtypeStruct((h.shape[0], w.shape[1]), jnp.float32),
            jax.ShapeDtypeStruct((1, 1), jnp.float32),
        ],
    )(h, w, b[None, :], hstack)


def kernel(x, pos, edge_index0, edge_index1, edge_index2, edge_index3,
           edge_index4, cluster1, cluster2, cluster3, cluster4, cluster5,
           params):
    eis = [edge_index0, edge_index1, edge_index2, edge_index3, edge_index4]
    clusters = [cluster1, cluster2, cluster3, cluster4, cluster5]
    houts = []
    table = x
    for l in range(5):
        p = params[f"conv{l + 1}"]
        x, hout = _conv(l, x, pos, eis[l], p, table)
        houts.append(hout)
        if l < 4:
            pn = params[f"conv{l + 2}"]
            ci_n, co_n = DIMS[l + 1]
            wkp = pn["Wk"].transpose(1, 0, 2).reshape(ci_n, K * co_n)
        else:
            wkp = None
        x, pos, table = _pool(l, x, pos, clusters[l], POOL_SIZES[l], wkp)
    h = x.reshape(-1, VOX * 47)
    ls, closs = _final_tc(h, params["fcW"], params["fcb"], jnp.stack(houts))
    return ls, closs.reshape(())


# speedup vs baseline: 8.8113x; 1.2980x over previous
import functools
import math

import jax
import jax.numpy as jnp
from jax import lax
from jax.experimental import pallas as pl
from jax.experimental.pallas import tpu as pltpu
from jax.experimental.pallas import tpu_sc as plsc

N0, N1, N2, N3, N4 = 100000, 25000, 6250, 1600, 400
B, VOX = 50, 8
K = 5
DIMS = [(1, 12), (15, 20), (23, 28), (31, 36), (39, 44)]
SIZES = [N0, N1, N2, N3, N4]
POOL_SIZES = [N1, N2, N3, N4, B * VOX]
DEG = 16

NC, NS, LANES = 2, 16, 16
NW = NC * NS
_INTERPRET = False

CONV_BN = [2000, 1000, 3128, 1600, 400]

CONV_CFG = [
    dict(J=4, mode="outer"),
    dict(J=1, mode="ymsg"),
    dict(J=1, mode="ymsg"),
    dict(J=1, mode="ymsg"),
    dict(J=1, mode="ymsg"),
]

LN2 = 0.6931471805599453
SQRT2 = 1.4142135623730951


def _vlog(s):
    bits = plsc.bitcast(s, jnp.int32)
    e = (lax.shift_right_logical(bits, 23) & 0xFF) - 127
    m_bits = (bits & 0x7FFFFF) | 0x3F800000
    m = plsc.bitcast(m_bits, jnp.float32)
    big = m > SQRT2
    m = jnp.where(big, m * 0.5, m)
    ef = e.astype(jnp.float32) + jnp.where(big, 1.0, 0.0)
    t = (m - 1.0) / (m + 1.0)
    t2 = t * t
    p = 1.0 + t2 * (1.0 / 3.0 + t2 * (1.0 / 5.0 + t2 * (1.0 / 7.0)))
    return ef * LN2 + 2.0 * t * p


def _conv_sc_level(level, n, ci, co, e_real):
    cfg = CONV_CFG[level]
    J = cfg["J"]
    mode = cfg["mode"]
    C = J * 128
    TW = ci if mode == "outer" else K * co
    TWp = math.ceil(TW / 16) * 16
    w_real = (K * ci + 1) if mode == "outer" else (co + 1)
    W = math.ceil(w_real / 8) * 8
    mw = math.ceil(e_real / (NW * C))
    if mw % 2:
        mw += 1
    e_pad = NW * C * mw
    n_pad = math.ceil(n / 8) * 8
    kc = math.ceil(n_pad / (NS * 128))

    mesh = plsc.VectorSubcoreMesh(
        core_axis_name="c", subcore_axis_name="s", num_cores=NC, num_subcores=NS)

    def set_types():
        t = [
            pltpu.VMEM((J, 128), jnp.int32),
            pltpu.VMEM((J, 128), jnp.int32),
            pltpu.VMEM((C, 16), jnp.float32),
            pltpu.VMEM((C, 16), jnp.float32),
            pltpu.VMEM((C, TWp), jnp.float32),
            pltpu.VMEM((C, W), jnp.float32),
        ]
        for _ in range(J):
            t.append(pltpu.VMEM((128,), jnp.int32))
        t.append(pltpu.SemaphoreType.DMA)
        return t

    nset = 6 + J + 1
    scratch = [pltpu.VMEM_SHARED((n_pad, W), jnp.float32)]
    scratch += set_types() + set_types()
    scratch += [
        pltpu.VMEM((64, W), jnp.float32),
        pltpu.VMEM((16,), jnp.float32),
        pltpu.VMEM((16,), jnp.float32),
        pltpu.SemaphoreType.DMA,
    ]

    def body(src_hbm, dst_hbm, tab_hbm, pos_hbm, bmap_hbm, z_hbm, s2_out, h_out,
             *scr):
        s_sh = scr[0]
        sets = []
        for si in range(2):
            base = 1 + si * nset
            sets.append(dict(
                sidx2=scr[base], didx2=scr[base + 1], ps2=scr[base + 2],
                pd2=scr[base + 3], tg2=scr[base + 4], rows2=scr[base + 5],
                didx=list(scr[base + 6:base + 6 + J]), sem=scr[base + 6 + J]))
        zbuf, hbuf, bmap_v, sem_s = scr[1 + 2 * nset:1 + 2 * nset + 4]

        cid = lax.axis_index("c")
        sid = lax.axis_index("s")
        wid = sid * NC + cid

        pltpu.sync_copy(bmap_hbm, bmap_v)
        pltpu.sync_copy(z_hbm, zbuf)
        n_zc = math.ceil(n_pad / (NS * 64))
        zrows = n_zc * 64

        def _zs(i, _):
            r0 = jnp.minimum(sid * zrows + i * 64, n_pad - 64)
            r0 = jnp.maximum(r0, 0)
            pltpu.sync_copy(zbuf, s_sh.at[pl.ds(r0, 64)])
            return 0
        lax.fori_loop(0, n_zc, _zs, 0)
        plsc.subcore_barrier()

        iota = lax.iota(jnp.int32, 16)
        bvec = bmap_v[...]
        bm = [[bvec[i * K + k] for k in range(K)] for i in range(3)]

        def col(c):
            return jnp.full((16,), c, jnp.int32)

        def load_idx(m, st):
            row = m * J
            pltpu.sync_copy(src_hbm.at[pl.ds(row, J)], st["sidx2"])
            pltpu.sync_copy(dst_hbm.at[pl.ds(row, J)], st["didx2"])

        def fire_gathers(st):
            cps = []
            for j in range(J):
                cps.append(pltpu.async_copy(
                    pos_hbm.at[st["sidx2"].at[j]],
                    st["ps2"].at[pl.ds(j * 128, 128)], st["sem"]))
                cps.append(pltpu.async_copy(
                    pos_hbm.at[st["didx2"].at[j]],
                    st["pd2"].at[pl.ds(j * 128, 128)], st["sem"]))
                cps.append(pltpu.async_copy(
                    tab_hbm.at[st["sidx2"].at[j]],
                    st["tg2"].at[pl.ds(j * 128, 128)], st["sem"]))
            return cps

        def redistribute(st):
            for j in range(J):
                for v in range(8):
                    st["didx"][j][pl.ds(v * 16, 16)] = st["didx2"][
                        j, pl.ds(v * 16, 16)]

        def compute(m, st, hacc):
            chunk_off = m * C
            ps2, pd2, tg2, rows2 = st["ps2"], st["pd2"], st["tg2"], st["rows2"]
            for j in range(J):
                def v_body(v8, h_in, j=j):
                    rl = v8 * 16 + iota
                    fr = j * 128 + rl
                    eg = chunk_off + fr
                    vf = jnp.where(eg < e_real, 1.0, 0.0)
                    p0 = plsc.load_gather(ps2, [fr, col(0)])
                    p1 = plsc.load_gather(ps2, [fr, col(1)])
                    p2 = plsc.load_gather(ps2, [fr, col(2)])
                    q0 = plsc.load_gather(pd2, [fr, col(0)])
                    q1 = plsc.load_gather(pd2, [fr, col(1)])
                    q2 = plsc.load_gather(pd2, [fr, col(2)])
                    u0, u1, u2 = q0 - p0, q1 - p1, q2 - p2
                    z = [u0 * bm[0][k] + u1 * bm[1][k] + u2 * bm[2][k]
                         for k in range(K)]
                    zm = z[0]
                    for k in range(1, K):
                        zm = jnp.maximum(zm, z[k])
                    ez = [jnp.exp(zk - zm) for zk in z]
                    ssum = ez[0]
                    for k in range(1, K):
                        ssum = ssum + ez[k]
                    inv = 1.0 / ssum
                    alpha = [ek * inv for ek in ez]
                    dot = alpha[0] * (z[0] - zm)
                    for k in range(1, K):
                        dot = dot + alpha[k] * (z[k] - zm)
                    h_new = h_in + vf * (_vlog(ssum) - dot)
                    if mode == "outer":
                        xs = [plsc.load_gather(tg2, [fr, col(i)])
                              for i in range(ci)]
                        for k in range(K):
                            avk = alpha[k] * vf
                            for i in range(ci):
                                plsc.store_scatter(
                                    rows2, [fr, col(k * ci + i)], avk * xs[i])
                    else:
                        for o in range(co):
                            acc = alpha[0] * plsc.load_gather(tg2, [fr, col(o)])
                            for k in range(1, K):
                                acc = acc + alpha[k] * plsc.load_gather(
                                    tg2, [fr, col(k * co + o)])
                            plsc.store_scatter(rows2, [fr, col(o)], acc * vf)
                    plsc.store_scatter(rows2, [fr, col(w_real - 1)], vf)
                    for pc in range(w_real, W):
                        plsc.store_scatter(rows2, [fr, col(pc)],
                                           jnp.zeros((16,), jnp.float32))
                    return h_new
                hacc = lax.fori_loop(0, 8, v_body, hacc)
            return hacc

        def fire_scatters(st):
            cps = []
            for j in range(J):
                cps.append(pltpu.async_copy(
                    st["rows2"].at[pl.ds(j * 128, 128)],
                    s_sh.at[st["didx"][j]], sem_s, add=True))
            return cps

        def super_body(m2, hacc):
            a = (wid * mw + 2 * m2)
            b = a + 1
            sa, sb = sets[0], sets[1]
            load_idx(a, sa)
            ga = fire_gathers(sa)
            load_idx(b, sb)
            gb = fire_gathers(sb)
            redistribute(sa)
            for cp in ga:
                cp.wait()
            hacc = compute(a, sa, hacc)
            sca = fire_scatters(sa)
            redistribute(sb)
            for cp in gb:
                cp.wait()
            hacc = compute(b, sb, hacc)
            for cp in sca:
                cp.wait()
            scb = fire_scatters(sb)
            for cp in scb:
                cp.wait()
            return hacc

        hacc = lax.fori_loop(0, mw // 2, super_body,
                             jnp.zeros((16,), jnp.float32))
        hbuf[...] = hacc
        pltpu.sync_copy(hbuf, h_out.at[wid])
        plsc.subcore_barrier()

        def _out(i, _):
            r0 = jnp.minimum(sid * (kc * 128) + i * 128, n_pad - 128)
            pltpu.sync_copy(s_sh.at[pl.ds(r0, 128)], s2_out.at[cid, pl.ds(r0, 128)])
            return 0
        lax.fori_loop(0, kc, _out, 0)

    kern = pl.kernel(
        body,
        out_type=[
            jax.ShapeDtypeStruct((2, n_pad, W), jnp.float32),
            jax.ShapeDtypeStruct((NW, 16), jnp.float32),
        ],
        mesh=mesh,
        scratch_types=scratch,
        compiler_params=pltpu.CompilerParams(
            needs_layout_passes=False, use_tc_tiling_on_sc=False),
        interpret=_INTERPRET,
    )

    def run(ei, table, pos, bmap):
        srcF = jnp.pad(ei[0], (0, e_pad - e_real)).reshape(e_pad // 128, 128)
        dstF = jnp.pad(ei[1], (0, e_pad - e_real)).reshape(e_pad // 128, 128)
        tpad = jnp.pad(table, ((0, 0), (0, TWp - TW)))
        ppad = jnp.pad(pos, ((0, 0), (0, 13)))
        bmap16 = jnp.pad(bmap.reshape(15), (0, 1))
        zeros = jnp.zeros((64, W), jnp.float32)
        return kern(srcF, dstF, tpad, ppad, bmap16, zeros)

    return run


_CONV_RUNNERS = {}


def _get_conv_runner(level, n, ci, co, e_real):
    key = (level, n, ci, co, e_real)
    if key not in _CONV_RUNNERS:
        _CONV_RUNNERS[key] = _conv_sc_level(level, n, ci, co, e_real)
    return _CONV_RUNNERS[key]


def _conv_tc_post(mode, s2, x, wk2, wroot, bias, bn):
    n, ci = x.shape
    co = wroot.shape[1]
    n_pad = s2.shape[1]
    x = jnp.pad(x, ((0, n_pad - n), (0, 0)))
    w = s2.shape[2]
    grid = n_pad // bn

    def body(s2_ref, x_ref, wk2_ref, wroot_ref, bias_ref, o_ref):
        s = s2_ref[0] + s2_ref[1]
        if mode == "outer":
            agg = jnp.dot(s[:, :K * ci], wk2_ref[...],
                          preferred_element_type=jnp.float32)
            cnt = s[:, K * ci]
        else:
            agg = s[:, :co]
            cnt = s[:, co]
        agg = agg / jnp.maximum(cnt, 1.0)[:, None]
        out = jnp.dot(x_ref[...], wroot_ref[...],
                      preferred_element_type=jnp.float32)
        out = out + agg + bias_ref[0:1]
        o_ref[...] = jnp.where(out > 0, out, jnp.exp(jnp.minimum(out, 0.0)) - 1.0)

    res = pl.pallas_call(
        body,
        grid=(grid,),
        in_specs=[
            pl.BlockSpec((2, bn, w), lambda i: (0, i, 0)),
            pl.BlockSpec((bn, ci), lambda i: (i, 0)),
            pl.BlockSpec(wk2.shape, lambda i: (0, 0)),
            pl.BlockSpec(wroot.shape, lambda i: (0, 0)),
            pl.BlockSpec((8, co), lambda i: (0, 0)),
        ],
        out_specs=pl.BlockSpec((bn, co), lambda i: (i, 0)),
        out_shape=jax.ShapeDtypeStruct((n_pad, co), jnp.float32),
    )(s2, x, wk2, wroot, jnp.broadcast_to(bias[None, :], (8, co)))
    return res[:n]


def _conv(level, x, pos, ei, p, table):
    n, ci = x.shape
    co = p["Wroot"].shape[1]
    e_real = ei.shape[1]
    mode = CONV_CFG[level]["mode"]
    run = _get_conv_runner(level, n, ci, co, e_real)
    s2, hout = run(ei, table, pos, p["Bmap"])
    wk2 = p["Wk"].reshape(K * ci, co)
    out = _conv_tc_post(mode, s2, x, wk2, p["Wroot"], p["bias"], CONV_BN[level])
    return out, hout


POOL_CFG = [
    dict(G=8, R=4, Jp=4),
    dict(G=16, R=2, Jp=4),
    dict(G=16, R=2, Jp=2),
    dict(G=32, R=1, Jp=1),
    dict(G=32, R=1, Jp=1),
]

NEG_BIG = -3.0e38


def _pool_sc_level(level, nn, s_out, f):
    cfg = POOL_CFG[level]
    G, R, Jp = cfg["G"], cfg["R"], cfg["Jp"]
    Cp = Jp * 128
    s_pad = math.ceil(s_out / 8) * 8
    rs = math.ceil(math.ceil(s_pad / R) / 8) * 8
    mp = math.ceil(nn / (G * Cp))
    kc4 = math.ceil(s_pad / (NS * 128))

    mesh = plsc.VectorSubcoreMesh(
        core_axis_name="c", subcore_axis_name="s", num_cores=NC, num_subcores=NS)

    scratch = [
        pltpu.VMEM_SHARED((s_pad, 8), jnp.float32),
        pltpu.VMEM((rs, f), jnp.float32),
        pltpu.VMEM((Cp,), jnp.int32),
        pltpu.VMEM((Cp, f), jnp.float32),
        pltpu.VMEM((Cp, 3), jnp.float32),
        pltpu.VMEM((Cp, 8), jnp.float32),
        pltpu.VMEM((64, 8), jnp.float32),
    ]
    for _ in range(Jp):
        scratch.append(pltpu.VMEM((128,), jnp.int32))
    scratch.append(pltpu.SemaphoreType.DMA)

    def body(x_hbm, pos_hbm, cl_hbm, neg_hbm, z4_hbm, xp_out, p2_out, *scr):
        p_sh, table, clb, xb, posb, rows4, z4buf = scr[:7]
        clidx = list(scr[7:7 + Jp])
        sem = scr[-1]

        cid = lax.axis_index("c")
        sid = lax.axis_index("s")
        wid = sid * NC + cid
        g = wid // R
        r = wid - g * R
        rbase = r * rs

        n_tc = math.ceil(rs / 64)
        pltpu.sync_copy(z4_hbm, z4buf)

        def _ti(i, _):
            r0 = jnp.maximum(jnp.minimum(i * 64, rs - 64), 0)
            pltpu.sync_copy(neg_hbm, table.at[pl.ds(r0, 64)])
            return 0
        lax.fori_loop(0, n_tc, _ti, 0)

        n_zc = math.ceil(s_pad / (NS * 64))
        zrows = n_zc * 64

        def _zs(i, _):
            r0 = jnp.maximum(jnp.minimum(sid * zrows + i * 64, s_pad - 64), 0)
            pltpu.sync_copy(z4buf, p_sh.at[pl.ds(r0, 64)])
            return 0
        lax.fori_loop(0, n_zc, _zs, 0)
        plsc.subcore_barrier()

        iota = lax.iota(jnp.int32, 16)

        def col(c):
            return jnp.full((16,), c, jnp.int32)

        def chunk_body(m, _):
            off = (g * mp + m) * Cp
            xcp = pltpu.async_copy(x_hbm.at[pl.ds(off, Cp)], xb, sem)
            pltpu.sync_copy(cl_hbm.at[pl.ds(off, Cp)], clb)
            xcp.wait()

            for j in range(Jp):
                def v_body(v8, _, j=j):
                    rl = j * 128 + v8 * 16 + iota
                    nidx = off + rl
                    valid = nidx < nn
                    cl = clb[pl.ds(j * 128 + v8 * 16, 16)]
                    loc = cl - rbase
                    msk = valid & (loc >= 0) & (loc < rs)
                    lcl = jnp.clip(loc, 0, rs - 1)
                    for ff in range(f):
                        xv = plsc.load_gather(xb, [rl, col(ff)])

                        def cond(carry):
                            return jnp.max(carry.astype(jnp.int32)) > 0

                        def it(carry):
                            cur = plsc.load_gather(table, [lcl, col(ff)])
                            new = jnp.maximum(cur, xv)
                            plsc.store_scatter(table, [lcl, col(ff)], new,
                                               mask=carry)
                            chk = plsc.load_gather(table, [lcl, col(ff)])
                            return carry & (chk < xv)

                        lax.while_loop(cond, it, msk)
                    return 0
                lax.fori_loop(0, 8, v_body, 0)

            @pl.when(r == 0)
            def _pos_part():
                pltpu.sync_copy(pos_hbm.at[pl.ds(off, Cp)], posb)
                for j in range(Jp):
                    def p_body(v8, _, j=j):
                        rl = j * 128 + v8 * 16 + iota
                        vf = jnp.where(off + rl < nn, 1.0, 0.0)
                        for cc in range(3):
                            pv = plsc.load_gather(posb, [rl, col(cc)])
                            plsc.store_scatter(rows4, [rl, col(cc)], pv * vf)
                        plsc.store_scatter(rows4, [rl, col(3)], vf)
                        for pc in range(4, 8):
                            plsc.store_scatter(rows4, [rl, col(pc)],
                                               jnp.zeros((16,), jnp.float32))
                        return 0
                    lax.fori_loop(0, 8, p_body, 0)
                for j in range(Jp):
                    for v in range(8):
                        clidx[j][pl.ds(v * 16, 16)] = clb[
                            pl.ds(j * 128 + v * 16, 16)]
                scs = [pltpu.async_copy(rows4.at[pl.ds(j * 128, 128)],
                                        p_sh.at[clidx[j]], sem, add=True)
                       for j in range(Jp)]
                for cp in scs:
                    cp.wait()
            return 0

        lax.fori_loop(0, mp, chunk_body, 0)
        plsc.subcore_barrier()

        n_oc = math.ceil(rs / 128)

        def _ox(i, _):
            lo = jnp.minimum(i * 128, rs - 128)
            lo = jnp.minimum(lo, s_pad - 128 - rbase)
            lo = jnp.maximum(lo, 0)
            pltpu.sync_copy(table.at[pl.ds(lo, 128)],
                            xp_out.at[g, pl.ds(rbase + lo, 128)])
            return 0
        lax.fori_loop(0, n_oc, _ox, 0)

        def _op(i, _):
            r0 = jnp.maximum(jnp.minimum(sid * (kc4 * 128) + i * 128,
                                         s_pad - 128), 0)
            pltpu.sync_copy(p_sh.at[pl.ds(r0, 128)],
                            p2_out.at[cid, pl.ds(r0, 128)])
            return 0
        lax.fori_loop(0, kc4, _op, 0)

    kern = pl.kernel(
        body,
        out_type=[
            jax.ShapeDtypeStruct((G, s_pad, f), jnp.float32),
            jax.ShapeDtypeStruct((2, s_pad, 8), jnp.float32),
        ],
        mesh=mesh,
        scratch_types=scratch,
        compiler_params=pltpu.CompilerParams(
            needs_layout_passes=False, use_tc_tiling_on_sc=False),
        interpret=_INTERPRET,
    )

    def run(x, pos, cl):
        nn_pad = G * mp * Cp
        xp_ = jnp.pad(x, ((0, nn_pad - nn), (0, 0)))
        posp_ = jnp.pad(pos, ((0, nn_pad - nn), (0, 0)))
        clp_ = jnp.pad(cl, (0, nn_pad - nn))
        neg = jnp.full((64, f), NEG_BIG, jnp.float32)
        z4 = jnp.zeros((64, 8), jnp.float32)
        return kern(xp_, posp_, clp_, neg, z4)

    return run


_POOL_RUNNERS = {}


def _get_pool_runner(level, nn, s_out, f):
    key = (level, nn, s_out, f)
    if key not in _POOL_RUNNERS:
        _POOL_RUNNERS[key] = _pool_sc_level(level, nn, s_out, f)
    return _POOL_RUNNERS[key]


def _pool_tc_post(xp_g, p2, wkp, s, bn):
    g_, s_pad, f = xp_g.shape
    yw = 0 if wkp is None else wkp.shape[1]
    grid = s_pad // bn

    def body(*refs):
        if wkp is None:
            xp_ref, p2_ref, x_ref, pos_ref = refs
        else:
            xp_ref, p2_ref, wkp_ref, x_ref, pos_ref, y_ref = refs
        xp = xp_ref[0]
        for gg in range(1, g_):
            xp = jnp.maximum(xp, xp_ref[gg])
        xp = jnp.where(xp > NEG_BIG / 2, xp, 0.0)
        p = p2_ref[0] + p2_ref[1]
        cnt = p[:, 3]
        posp = p[:, :3] / jnp.maximum(cnt, 1.0)[:, None]
        xnew = jnp.concatenate([xp, posp], axis=1)
        x_ref[...] = xnew
        pos_ref[...] = posp
        if wkp is not None:
            y_ref[...] = jnp.dot(xnew, wkp_ref[...],
                                 preferred_element_type=jnp.float32)

    in_specs = [
        pl.BlockSpec((g_, bn, f), lambda i: (0, i, 0)),
        pl.BlockSpec((2, bn, 8), lambda i: (0, i, 0)),
    ]
    out_shape = [
        jax.ShapeDtypeStruct((s_pad, f + 3), jnp.float32),
        jax.ShapeDtypeStruct((s_pad, 3), jnp.float32),
    ]
    out_specs = [
        pl.BlockSpec((bn, f + 3), lambda i: (i, 0)),
        pl.BlockSpec((bn, 3), lambda i: (i, 0)),
    ]
    args = [xp_g, p2]
    if wkp is not None:
        in_specs.append(pl.BlockSpec(wkp.shape, lambda i: (0, 0)))
        out_shape.append(jax.ShapeDtypeStruct((s_pad, yw), jnp.float32))
        out_specs.append(pl.BlockSpec((bn, yw), lambda i: (i, 0)))
        args.append(wkp)
    res = pl.pallas_call(
        body, grid=(grid,), in_specs=in_specs, out_specs=out_specs,
        out_shape=out_shape,
    )(*args)
    xnew = res[0][:s]
    posnew = res[1][:s]
    ynext = res[2][:s] if wkp is not None else None
    return xnew, posnew, ynext


POOL_BN = [1000, 272, 400, 400, 400]


def _pool(level, x, pos, cluster, s, wkp):
    nn, f = x.shape
    run = _get_pool_runner(level, nn, s, f)
    xp_g, p2 = run(x, pos, cluster)
    return _pool_tc_post(xp_g, p2, wkp, s, POOL_BN[level])


def _final_tc(h, w, b, hstack):
    ecounts = [SIZES[l] * DEG for l in range(5)]

    def body(h_ref, w_ref, b_ref, hs_ref, ls_ref, closs_ref):
        logits = jnp.dot(h_ref[...], w_ref[...],
                         preferred_element_type=jnp.float32) + b_ref[...]
        m = jnp.max(logits, axis=1, keepdims=True)
        sh = logits - m
        lse = jnp.log(jnp.sum(jnp.exp(sh), axis=1, keepdims=True))
        ls_ref[...] = sh - lse
        c = jnp.float32(0.0)
        for l in range(5):
            c = c + jnp.sum(hs_ref[l]) / ecounts[l]
        closs_ref[...] = jnp.full((1, 1), 1.0, jnp.float32) * c

    return pl.pallas_call(
        body,
        out_shape=[
            jax.ShapeDtypeStruct((h.shape[0], w.shape[1]), jnp.float32),
            jax.ShapeDtypeStruct((1, 1), jnp.float32),
        ],
    )(h, w, b[None, :], hstack)


def kernel(x, pos, edge_index0, edge_index1, edge_index2, edge_index3,
           edge_index4, cluster1, cluster2, cluster3, cluster4, cluster5,
           params):
    eis = [edge_index0, edge_index1, edge_index2, edge_index3, edge_index4]
    clusters = [cluster1, cluster2, cluster3, cluster4, cluster5]
    houts = []
    table = x
    for l in range(5):
        p = params[f"conv{l + 1}"]
        x, hout = _conv(l, x, pos, eis[l], p, table)
        houts.append(hout)
        if l < 4:
            pn = params[f"conv{l + 2}"]
            ci_n, co_n = DIMS[l + 1]
            wkp = pn["Wk"].transpose(1, 0, 2).reshape(ci_n, K * co_n)
        else:
            wkp = None
        x, pos, table = _pool(l, x, pos, clusters[l], POOL_SIZES[l], wkp)
    h = x.reshape(-1, VOX * 47)
    ls, closs = _final_tc(h, params["fcW"], params["fcb"], jnp.stack(houts))
    return ls, closs.reshape(())
